# Initial kernel scaffold; baseline (speedup 1.0000x reference)
import jax
import jax.numpy as jnp
from jax.experimental import pallas as pl


def kernel(x_nchw, w1, w2):
    raise NotImplementedError("write your pallas kernel here")


# speedup vs baseline: 1.1603x; 1.1603x over previous
import functools
import math

import jax
import jax.numpy as jnp
from jax import lax
from jax.experimental import pallas as pl
from jax.experimental.pallas import tpu as pltpu

_G = 8
_GRID_MIN, _GRID_MAX = -2.0, 2.0
_INV_DENOM = (_G - 1) / (_GRID_MAX - _GRID_MIN)
_EPS = 1e-5
_GMID = _G // 2


def _fastkan_layer(x, w_ref, pad_ref, mlf, mrf, *, H, W, C, COUT, PADL):
    HW = H * W
    CG = C * _G
    CTOT = CG + C
    bdt = pad_ref.dtype

    silu = x * jax.nn.sigmoid(x)

    s1 = jnp.sum(x, axis=1, keepdims=True)
    s2 = jnp.sum(x * x, axis=1, keepdims=True)
    mean = s1 * (1.0 / HW)
    var = s2 * (1.0 / HW) - mean * mean
    xn = (x - mean) * lax.rsqrt(var + _EPS)

    v = jnp.clip((xn - _GRID_MIN) * _INV_DENOM - _GMID, -14.0, 14.0)
    t_up = jnp.exp(v + v)
    t_dn = jnp.exp(-(v + v))
    b_mid = jnp.exp(-(v * v))

    pad_ref[pl.ds(_GMID * C, C), pl.ds(PADL, HW)] = b_mid.astype(bdt)
    b = b_mid
    for k in range(1, _G - _GMID):
        b = (b * t_up) * math.exp(-(2 * k - 1))
        pad_ref[pl.ds((_GMID + k) * C, C), pl.ds(PADL, HW)] = b.astype(bdt)
    b = b_mid
    for k in range(1, _GMID + 1):
        b = (b * t_dn) * math.exp(-(2 * k - 1))
        pad_ref[pl.ds((_GMID - k) * C, C), pl.ds(PADL, HW)] = b.astype(bdt)

    pad_ref[pl.ds(CG, C), pl.ds(PADL, HW)] = silu.astype(bdt)

    accs = [jnp.zeros((COUT, HW), jnp.float32) for _ in range(3)]
    for ky in range(3):
        for kx in range(3):
            shift = (ky - 1) * W + (kx - 1)
            win = pad_ref[pl.ds(0, CTOT), pl.ds(PADL + shift, HW)]
            accs[kx] = accs[kx] + jnp.dot(w_ref[ky * 3 + kx], win,
                                          preferred_element_type=jnp.float32)
    return accs[1] + accs[0] * mlf + accs[2] * mrf


def _encoder_block_kernel(x_ref, w1_ref, w2_ref, o_ref, pad_ref,
                          *, H, W, C1, C2, PADL):
    HW = H * W
    CMAX = pad_ref.shape[0]

    hz = jnp.zeros((CMAX, W + 1), pad_ref.dtype)
    pad_ref[pl.ds(0, CMAX), pl.ds(PADL - (W + 1), W + 1)] = hz
    pad_ref[pl.ds(0, CMAX), pl.ds(PADL + HW, W + 1)] = hz

    col = lax.broadcasted_iota(jnp.int32, (1, HW), 1) % W
    mlf = (col >= 1).astype(jnp.float32)
    mrf = (col < (W - 1)).astype(jnp.float32)

    x = x_ref[0]
    y = _fastkan_layer(x, w1_ref, pad_ref, mlf, mrf,
                       H=H, W=W, C=C1, COUT=C2, PADL=PADL)
    z = _fastkan_layer(y, w2_ref, pad_ref, mlf, mrf,
                       H=H, W=W, C=C2, COUT=C2, PADL=PADL)
    o_ref[0] = z.astype(o_ref.dtype)


def _permute_to_g_major(w, c):
    cg = c * _G
    sp = w[:, :, :cg].reshape(9, -1, c, _G)
    sp = jnp.transpose(sp, (0, 1, 3, 2)).reshape(9, -1, cg)
    return jnp.concatenate([sp, w[:, :, cg:]], axis=-1).astype(jnp.bfloat16)


def kernel(x_nchw, w1, w2):
    n, c1, hh, ww = x_nchw.shape
    c2 = w1.shape[1]
    ct1, ct2 = w1.shape[2], w2.shape[2]
    hw = hh * ww
    padl = ((ww + 1 + 127) // 128) * 128
    lpad = padl + hw + ww + 1
    cmax = max(ct1, ct2)

    w1b = _permute_to_g_major(w1, c1)
    w2b = _permute_to_g_major(w2, c2)
    x_flat = x_nchw.reshape(n, c1, hw)

    body = functools.partial(_encoder_block_kernel,
                             H=hh, W=ww, C1=c1, C2=c2, PADL=padl)
    out = pl.pallas_call(
        body,
        out_shape=jax.ShapeDtypeStruct((n, c2, hw), jnp.float32),
        grid_spec=pltpu.PrefetchScalarGridSpec(
            num_scalar_prefetch=0,
            grid=(n,),
            in_specs=[
                pl.BlockSpec((1, c1, hw), lambda i: (i, 0, 0)),
                pl.BlockSpec((9, c2, ct1), lambda i: (0, 0, 0)),
                pl.BlockSpec((9, c2, ct2), lambda i: (0, 0, 0)),
            ],
            out_specs=pl.BlockSpec((1, c2, hw), lambda i: (i, 0, 0)),
            scratch_shapes=[pltpu.VMEM((cmax, lpad), jnp.bfloat16)],
        ),
        compiler_params=pltpu.CompilerParams(
            dimension_semantics=("parallel",)),
    )(x_flat, w1b, w2b)
    return out.reshape(n, c2, hh, ww)


# speedup vs baseline: 1.4263x; 1.2293x over previous
import functools
import math

import jax
import jax.numpy as jnp
from jax import lax
from jax.experimental import pallas as pl
from jax.experimental.pallas import tpu as pltpu

_G = 8
_GRID_MIN, _GRID_MAX = -2.0, 2.0
_INV_DENOM = (_G - 1) / (_GRID_MAX - _GRID_MIN)
_EPS = 1e-5
_GMID = _G // 2


def _fastkan_layer(x, w_ref, pad_ref, mlf, mrf, *, H, W, C, COUT, PADL):
    HW = H * W
    CG = C * _G
    CTOT = CG + C
    bdt = pad_ref.dtype

    silu = x * jax.nn.sigmoid(x)

    s1 = jnp.sum(x, axis=1, keepdims=True)
    s2 = jnp.sum(x * x, axis=1, keepdims=True)
    mean = s1 * (1.0 / HW)
    var = s2 * (1.0 / HW) - mean * mean
    xn = (x - mean) * lax.rsqrt(var + _EPS)

    v = jnp.clip((xn - _GRID_MIN) * _INV_DENOM - _GMID, -14.0, 14.0)
    t_up = jnp.exp(v + v)
    t_dn = jnp.exp(-(v + v))
    b_mid = jnp.exp(-(v * v))

    pad_ref[pl.ds(_GMID * C, C), pl.ds(PADL, HW)] = b_mid.astype(bdt)
    b = b_mid
    for k in range(1, _G - _GMID):
        b = (b * t_up) * math.exp(-(2 * k - 1))
        pad_ref[pl.ds((_GMID + k) * C, C), pl.ds(PADL, HW)] = b.astype(bdt)
    b = b_mid
    for k in range(1, _GMID + 1):
        b = (b * t_dn) * math.exp(-(2 * k - 1))
        pad_ref[pl.ds((_GMID - k) * C, C), pl.ds(PADL, HW)] = b.astype(bdt)

    pad_ref[pl.ds(CG, C), pl.ds(PADL, HW)] = silu.astype(bdt)

    data = pad_ref[pl.ds(0, CTOT), pl.ds(PADL - 128, HW + 256)]
    accs = [jnp.zeros((COUT, HW), jnp.float32) for _ in range(3)]
    for ky in range(3):
        for kx in range(3):
            shift = (ky - 1) * W + (kx - 1)
            p = jnp.dot(w_ref[ky * 3 + kx], data,
                        preferred_element_type=jnp.float32)
            accs[kx] = accs[kx] + p[:, 128 + shift:128 + shift + HW]
    return accs[1] + accs[0] * mlf + accs[2] * mrf


def _encoder_block_kernel(x_ref, w1_ref, w2_ref, o_ref, pad_ref,
                          *, H, W, C1, C2, PADL):
    HW = H * W
    CMAX = pad_ref.shape[0]

    hz = jnp.zeros((CMAX, W + 1), pad_ref.dtype)
    pad_ref[pl.ds(0, CMAX), pl.ds(PADL - (W + 1), W + 1)] = hz
    pad_ref[pl.ds(0, CMAX), pl.ds(PADL + HW, W + 1)] = hz

    col = lax.broadcasted_iota(jnp.int32, (1, HW), 1) % W
    mlf = (col >= 1).astype(jnp.float32)
    mrf = (col < (W - 1)).astype(jnp.float32)

    x = x_ref[0]
    y = _fastkan_layer(x, w1_ref, pad_ref, mlf, mrf,
                       H=H, W=W, C=C1, COUT=C2, PADL=PADL)
    z = _fastkan_layer(y, w2_ref, pad_ref, mlf, mrf,
                       H=H, W=W, C=C2, COUT=C2, PADL=PADL)
    o_ref[0] = z.astype(o_ref.dtype)


def _permute_to_g_major(w, c):
    cg = c * _G
    sp = w[:, :, :cg].reshape(9, -1, c, _G)
    sp = jnp.transpose(sp, (0, 1, 3, 2)).reshape(9, -1, cg)
    return jnp.concatenate([sp, w[:, :, cg:]], axis=-1).astype(jnp.bfloat16)


def kernel(x_nchw, w1, w2):
    n, c1, hh, ww = x_nchw.shape
    c2 = w1.shape[1]
    ct1, ct2 = w1.shape[2], w2.shape[2]
    hw = hh * ww
    padl = ((ww + 1 + 127) // 128) * 128
    lpad = padl + hw + 128
    cmax = max(ct1, ct2)

    w1b = _permute_to_g_major(w1, c1)
    w2b = _permute_to_g_major(w2, c2)
    x_flat = x_nchw.reshape(n, c1, hw)

    body = functools.partial(_encoder_block_kernel,
                             H=hh, W=ww, C1=c1, C2=c2, PADL=padl)
    out = pl.pallas_call(
        body,
        out_shape=jax.ShapeDtypeStruct((n, c2, hw), jnp.float32),
        grid_spec=pltpu.PrefetchScalarGridSpec(
            num_scalar_prefetch=0,
            grid=(n,),
            in_specs=[
                pl.BlockSpec((1, c1, hw), lambda i: (i, 0, 0)),
                pl.BlockSpec((9, c2, ct1), lambda i: (0, 0, 0)),
                pl.BlockSpec((9, c2, ct2), lambda i: (0, 0, 0)),
            ],
            out_specs=pl.BlockSpec((1, c2, hw), lambda i: (i, 0, 0)),
            scratch_shapes=[pltpu.VMEM((cmax, lpad), jnp.bfloat16)],
        ),
        compiler_params=pltpu.CompilerParams(
            dimension_semantics=("parallel",)),
    )(x_flat, w1b, w2b)
    return out.reshape(n, c2, hh, ww)


# speedup vs baseline: 2.1632x; 1.5167x over previous
import functools
import math

import jax
import jax.numpy as jnp
from jax import lax
from jax.experimental import pallas as pl
from jax.experimental.pallas import tpu as pltpu

_G = 8
_GRID_MIN, _GRID_MAX = -2.0, 2.0
_INV_DENOM = (_G - 1) / (_GRID_MAX - _GRID_MIN)
_EPS = 1e-5
_GMID = _G // 2


def _fastkan_layer(x, w_ref, pad_ref, mlf, mrf, *, H, W, C, COUT, PADL):
    HW = H * W
    CG = C * _G
    CTOT = CG + C
    bdt = pad_ref.dtype

    silu = x * jax.nn.sigmoid(x)

    s1 = jnp.sum(x, axis=1, keepdims=True)
    s2 = jnp.sum(x * x, axis=1, keepdims=True)
    mean = s1 * (1.0 / HW)
    var = s2 * (1.0 / HW) - mean * mean
    xn = (x - mean) * lax.rsqrt(var + _EPS)

    v = jnp.clip((xn - _GRID_MIN) * _INV_DENOM - _GMID, -14.0, 14.0)
    t_up = jnp.exp(v + v)
    t_dn = jnp.exp(-(v + v))
    b_mid = jnp.exp(-(v * v))

    pad_ref[pl.ds(_GMID * C, C), pl.ds(PADL, HW)] = b_mid.astype(bdt)
    b = b_mid
    for k in range(1, _G - _GMID):
        b = (b * t_up) * math.exp(-(2 * k - 1))
        pad_ref[pl.ds((_GMID + k) * C, C), pl.ds(PADL, HW)] = b.astype(bdt)
    b = b_mid
    for k in range(1, _GMID + 1):
        b = (b * t_dn) * math.exp(-(2 * k - 1))
        pad_ref[pl.ds((_GMID - k) * C, C), pl.ds(PADL, HW)] = b.astype(bdt)

    pad_ref[pl.ds(CG, C), pl.ds(PADL, HW)] = silu.astype(bdt)

    data = pad_ref[pl.ds(0, CTOT), pl.ds(PADL - 128, HW + 256)]
    p = jnp.dot(w_ref[...], data,
                preferred_element_type=jnp.float32)
    accs = [jnp.zeros((COUT, HW), jnp.float32) for _ in range(3)]
    for ky in range(3):
        for kx in range(3):
            r0 = (ky * 3 + kx) * COUT
            c0 = 128 + (ky - 1) * W + (kx - 1)
            accs[kx] = accs[kx] + p[r0:r0 + COUT, c0:c0 + HW]
    return accs[1] + accs[0] * mlf + accs[2] * mrf


def _encoder_block_kernel(x_ref, w1_ref, w2_ref, o_ref, pad_ref,
                          *, H, W, C1, C2, PADL):
    HW = H * W
    CMAX = pad_ref.shape[0]

    hz = jnp.zeros((CMAX, W + 1), pad_ref.dtype)
    pad_ref[pl.ds(0, CMAX), pl.ds(PADL - (W + 1), W + 1)] = hz
    pad_ref[pl.ds(0, CMAX), pl.ds(PADL + HW, W + 1)] = hz

    col = lax.broadcasted_iota(jnp.int32, (1, HW), 1) % W
    mlf = (col >= 1).astype(jnp.float32)
    mrf = (col < (W - 1)).astype(jnp.float32)

    x = x_ref[0]
    y = _fastkan_layer(x, w1_ref, pad_ref, mlf, mrf,
                       H=H, W=W, C=C1, COUT=C2, PADL=PADL)
    z = _fastkan_layer(y, w2_ref, pad_ref, mlf, mrf,
                       H=H, W=W, C=C2, COUT=C2, PADL=PADL)
    o_ref[0] = z.astype(o_ref.dtype)


def _permute_to_g_major(w, c):
    cg = c * _G
    sp = w[:, :, :cg].reshape(9, -1, c, _G)
    sp = jnp.transpose(sp, (0, 1, 3, 2)).reshape(9, -1, cg)
    wp = jnp.concatenate([sp, w[:, :, cg:]], axis=-1).astype(jnp.bfloat16)
    return wp.reshape(-1, wp.shape[-1])


def kernel(x_nchw, w1, w2):
    n, c1, hh, ww = x_nchw.shape
    c2 = w1.shape[1]
    ct1, ct2 = w1.shape[2], w2.shape[2]
    hw = hh * ww
    padl = ((ww + 1 + 127) // 128) * 128
    lpad = padl + hw + 128
    cmax = max(ct1, ct2)

    w1b = _permute_to_g_major(w1, c1)
    w2b = _permute_to_g_major(w2, c2)
    x_flat = x_nchw.reshape(n, c1, hw)

    body = functools.partial(_encoder_block_kernel,
                             H=hh, W=ww, C1=c1, C2=c2, PADL=padl)
    out = pl.pallas_call(
        body,
        out_shape=jax.ShapeDtypeStruct((n, c2, hw), jnp.float32),
        grid_spec=pltpu.PrefetchScalarGridSpec(
            num_scalar_prefetch=0,
            grid=(n,),
            in_specs=[
                pl.BlockSpec((1, c1, hw), lambda i: (i, 0, 0)),
                pl.BlockSpec((9 * c2, ct1), lambda i: (0, 0)),
                pl.BlockSpec((9 * c2, ct2), lambda i: (0, 0)),
            ],
            out_specs=pl.BlockSpec((1, c2, hw), lambda i: (i, 0, 0)),
            scratch_shapes=[pltpu.VMEM((cmax, lpad), jnp.bfloat16)],
        ),
        compiler_params=pltpu.CompilerParams(
            dimension_semantics=("parallel",)),
    )(x_flat, w1b, w2b)
    return out.reshape(n, c2, hh, ww)


# speedup vs baseline: 2.5319x; 1.1704x over previous
import functools
import math

import jax
import jax.numpy as jnp
from jax import lax
from jax.experimental import pallas as pl
from jax.experimental.pallas import tpu as pltpu

_G = 8
_GRID_MIN, _GRID_MAX = -2.0, 2.0
_INV_DENOM = (_G - 1) / (_GRID_MAX - _GRID_MIN)
_EPS = 1e-5
_GMID = _G // 2


def _fastkan_layer(x, w_ref, pad_ref, mlf, mrf, *, H, W, C, COUT, PADL):
    HW = H * W
    CG = C * _G
    CTOT = CG + C
    bdt = pad_ref.dtype

    silu = x * jax.nn.sigmoid(x)

    s1 = jnp.sum(x, axis=1, keepdims=True)
    s2 = jnp.sum(x * x, axis=1, keepdims=True)
    mean = s1 * (1.0 / HW)
    var = s2 * (1.0 / HW) - mean * mean
    xn = (x - mean) * lax.rsqrt(var + _EPS)

    v = jnp.clip((xn - _GRID_MIN) * _INV_DENOM - _GMID, -14.0, 14.0)
    t_up = jnp.exp(v + v)
    t_dn = jnp.exp(-(v + v))
    b_mid = jnp.exp(-(v * v))

    pad_ref[pl.ds(_GMID * C, C), pl.ds(PADL, HW)] = b_mid.astype(bdt)
    b = b_mid
    for k in range(1, _G - _GMID):
        b = (b * t_up) * math.exp(-(2 * k - 1))
        pad_ref[pl.ds((_GMID + k) * C, C), pl.ds(PADL, HW)] = b.astype(bdt)
    b = b_mid
    for k in range(1, _GMID + 1):
        b = (b * t_dn) * math.exp(-(2 * k - 1))
        pad_ref[pl.ds((_GMID - k) * C, C), pl.ds(PADL, HW)] = b.astype(bdt)

    pad_ref[pl.ds(CG, C), pl.ds(PADL, HW)] = silu.astype(bdt)

    data = pad_ref[pl.ds(0, CTOT), pl.ds(PADL - 128, HW + 256)]
    p = jnp.dot(w_ref[...], data,
                preferred_element_type=jnp.float32)
    accs = [jnp.zeros((COUT, HW), jnp.float32) for _ in range(3)]
    for ky in range(3):
        for kx in range(3):
            r0 = (ky * 3 + kx) * COUT
            c0 = 128 + (ky - 1) * W + (kx - 1)
            accs[kx] = accs[kx] + p[r0:r0 + COUT, c0:c0 + HW]
    return accs[1] + accs[0] * mlf + accs[2] * mrf


def _encoder_block_kernel(x_ref, w1_ref, w2_ref, o_ref, pad_a, pad_b,
                          *, H, W, C1, C2, PADL):
    HW = H * W
    CMAX = pad_a.shape[0]

    hz = jnp.zeros((CMAX, W + 1), pad_a.dtype)
    for ref in (pad_a, pad_b):
        ref[pl.ds(0, CMAX), pl.ds(PADL - (W + 1), W + 1)] = hz
        ref[pl.ds(0, CMAX), pl.ds(PADL + HW, W + 1)] = hz

    col = lax.broadcasted_iota(jnp.int32, (1, HW), 1) % W
    mlf = (col >= 1).astype(jnp.float32)
    mrf = (col < (W - 1)).astype(jnp.float32)

    y_a = _fastkan_layer(x_ref[0], w1_ref, pad_a, mlf, mrf,
                         H=H, W=W, C=C1, COUT=C2, PADL=PADL)
    y_b = _fastkan_layer(x_ref[1], w1_ref, pad_b, mlf, mrf,
                         H=H, W=W, C=C1, COUT=C2, PADL=PADL)
    z_a = _fastkan_layer(y_a, w2_ref, pad_a, mlf, mrf,
                         H=H, W=W, C=C2, COUT=C2, PADL=PADL)
    z_b = _fastkan_layer(y_b, w2_ref, pad_b, mlf, mrf,
                         H=H, W=W, C=C2, COUT=C2, PADL=PADL)
    o_ref[0] = z_a.astype(o_ref.dtype)
    o_ref[1] = z_b.astype(o_ref.dtype)


def _permute_to_g_major(w, c):
    cg = c * _G
    sp = w[:, :, :cg].reshape(9, -1, c, _G)
    sp = jnp.transpose(sp, (0, 1, 3, 2)).reshape(9, -1, cg)
    wp = jnp.concatenate([sp, w[:, :, cg:]], axis=-1).astype(jnp.bfloat16)
    return wp.reshape(-1, wp.shape[-1])


def kernel(x_nchw, w1, w2):
    n, c1, hh, ww = x_nchw.shape
    c2 = w1.shape[1]
    ct1, ct2 = w1.shape[2], w2.shape[2]
    hw = hh * ww
    padl = ((ww + 1 + 127) // 128) * 128
    lpad = padl + hw + 128
    cmax = max(ct1, ct2)

    w1b = _permute_to_g_major(w1, c1)
    w2b = _permute_to_g_major(w2, c2)
    x_flat = x_nchw.reshape(n, c1, hw)

    body = functools.partial(_encoder_block_kernel,
                             H=hh, W=ww, C1=c1, C2=c2, PADL=padl)
    out = pl.pallas_call(
        body,
        out_shape=jax.ShapeDtypeStruct((n, c2, hw), jnp.float32),
        grid_spec=pltpu.PrefetchScalarGridSpec(
            num_scalar_prefetch=0,
            grid=(n // 2,),
            in_specs=[
                pl.BlockSpec((2, c1, hw), lambda i: (i, 0, 0)),
                pl.BlockSpec((9 * c2, ct1), lambda i: (0, 0)),
                pl.BlockSpec((9 * c2, ct2), lambda i: (0, 0)),
            ],
            out_specs=pl.BlockSpec((2, c2, hw), lambda i: (i, 0, 0)),
            scratch_shapes=[pltpu.VMEM((cmax, lpad), jnp.bfloat16),
                            pltpu.VMEM((cmax, lpad), jnp.bfloat16)],
        ),
        compiler_params=pltpu.CompilerParams(
            dimension_semantics=("parallel",)),
    )(x_flat, w1b, w2b)
    return out.reshape(n, c2, hh, ww)


# speedup vs baseline: 2.5783x; 1.0184x over previous
import functools
import math

import jax
import jax.numpy as jnp
from jax import lax
from jax.experimental import pallas as pl
from jax.experimental.pallas import tpu as pltpu

_G = 8
_GRID_MIN, _GRID_MAX = -2.0, 2.0
_INV_DENOM = (_G - 1) / (_GRID_MAX - _GRID_MIN)
_EPS = 1e-5
_GMID = _G // 2
_IPS = 4


def _fastkan_layer(x, w_ref, pad_ref, mlf, mrf, *, H, W, C, COUT, PADL):
    HW = H * W
    CG = C * _G
    CTOT = CG + C
    bdt = pad_ref.dtype

    silu = x * jax.nn.sigmoid(x)

    s1 = jnp.sum(x, axis=1, keepdims=True)
    s2 = jnp.sum(x * x, axis=1, keepdims=True)
    mean = s1 * (1.0 / HW)
    var = s2 * (1.0 / HW) - mean * mean
    xn = (x - mean) * lax.rsqrt(var + _EPS)

    v = jnp.clip((xn - _GRID_MIN) * _INV_DENOM - _GMID, -14.0, 14.0)
    t_up = jnp.exp(v + v)
    t_dn = jnp.exp(-(v + v))
    b_mid = jnp.exp(-(v * v))

    pad_ref[pl.ds(_GMID * C, C), pl.ds(PADL, HW)] = b_mid.astype(bdt)
    b = b_mid
    for k in range(1, _G - _GMID):
        b = (b * t_up) * math.exp(-(2 * k - 1))
        pad_ref[pl.ds((_GMID + k) * C, C), pl.ds(PADL, HW)] = b.astype(bdt)
    b = b_mid
    for k in range(1, _GMID + 1):
        b = (b * t_dn) * math.exp(-(2 * k - 1))
        pad_ref[pl.ds((_GMID - k) * C, C), pl.ds(PADL, HW)] = b.astype(bdt)

    pad_ref[pl.ds(CG, C), pl.ds(PADL, HW)] = silu.astype(bdt)

    data = pad_ref[pl.ds(0, CTOT), pl.ds(PADL - 128, HW + 256)]
    p = jnp.dot(w_ref[...], data,
                preferred_element_type=jnp.float32)
    accs = [jnp.zeros((COUT, HW), jnp.float32) for _ in range(3)]
    for ky in range(3):
        for kx in range(3):
            r0 = (ky * 3 + kx) * COUT
            c0 = 128 + (ky - 1) * W + (kx - 1)
            accs[kx] = accs[kx] + p[r0:r0 + COUT, c0:c0 + HW]
    return accs[1] + accs[0] * mlf + accs[2] * mrf


def _encoder_block_kernel(x_ref, w1_ref, w2_ref, o_ref, *pads,
                          H, W, C1, C2, PADL):
    HW = H * W
    CMAX = pads[0].shape[0]

    hz = jnp.zeros((CMAX, W + 1), pads[0].dtype)
    for ref in pads:
        ref[pl.ds(0, CMAX), pl.ds(PADL - (W + 1), W + 1)] = hz
        ref[pl.ds(0, CMAX), pl.ds(PADL + HW, W + 1)] = hz

    col = lax.broadcasted_iota(jnp.int32, (1, HW), 1) % W
    mlf = (col >= 1).astype(jnp.float32)
    mrf = (col < (W - 1)).astype(jnp.float32)

    ys = [_fastkan_layer(x_ref[i], w1_ref, pad, mlf, mrf,
                         H=H, W=W, C=C1, COUT=C2, PADL=PADL)
          for i, pad in enumerate(pads)]
    zs = [_fastkan_layer(y, w2_ref, pad, mlf, mrf,
                         H=H, W=W, C=C2, COUT=C2, PADL=PADL)
          for y, pad in zip(ys, pads)]
    for i, z in enumerate(zs):
        o_ref[i] = z.astype(o_ref.dtype)


def _permute_to_g_major(w, c):
    cg = c * _G
    sp = w[:, :, :cg].reshape(9, -1, c, _G)
    sp = jnp.transpose(sp, (0, 1, 3, 2)).reshape(9, -1, cg)
    wp = jnp.concatenate([sp, w[:, :, cg:]], axis=-1).astype(jnp.bfloat16)
    return wp.reshape(-1, wp.shape[-1])


def kernel(x_nchw, w1, w2):
    n, c1, hh, ww = x_nchw.shape
    c2 = w1.shape[1]
    ct1, ct2 = w1.shape[2], w2.shape[2]
    hw = hh * ww
    padl = ((ww + 1 + 127) // 128) * 128
    lpad = padl + hw + 128
    cmax = max(ct1, ct2)

    w1b = _permute_to_g_major(w1, c1)
    w2b = _permute_to_g_major(w2, c2)
    x_flat = x_nchw.reshape(n, c1, hw)

    body = functools.partial(_encoder_block_kernel,
                             H=hh, W=ww, C1=c1, C2=c2, PADL=padl)
    out = pl.pallas_call(
        body,
        out_shape=jax.ShapeDtypeStruct((n, c2, hw), jnp.float32),
        grid_spec=pltpu.PrefetchScalarGridSpec(
            num_scalar_prefetch=0,
            grid=(n // _IPS,),
            in_specs=[
                pl.BlockSpec((_IPS, c1, hw), lambda i: (i, 0, 0)),
                pl.BlockSpec((9 * c2, ct1), lambda i: (0, 0)),
                pl.BlockSpec((9 * c2, ct2), lambda i: (0, 0)),
            ],
            out_specs=pl.BlockSpec((_IPS, c2, hw), lambda i: (i, 0, 0)),
            scratch_shapes=[pltpu.VMEM((cmax, lpad), jnp.bfloat16)
                            for _ in range(_IPS)],
        ),
        compiler_params=pltpu.CompilerParams(
            dimension_semantics=("parallel",)),
    )(x_flat, w1b, w2b)
    return out.reshape(n, c2, hh, ww)


# speedup vs baseline: 2.5888x; 1.0041x over previous
import functools
import math

import jax
import jax.numpy as jnp
from jax import lax
from jax.experimental import pallas as pl
from jax.experimental.pallas import tpu as pltpu

_G = 8
_GRID_MIN, _GRID_MAX = -2.0, 2.0
_INV_DENOM = (_G - 1) / (_GRID_MAX - _GRID_MIN)
_EPS = 1e-5
_GMID = _G // 2


def _fastkan_layer(x, w_ref, pad_ref, mlf, mrf, *, H, W, C, COUT, PADL):
    HW = H * W
    CG = C * _G
    CTOT = CG + C
    bdt = pad_ref.dtype

    silu = x * jax.nn.sigmoid(x)

    s1 = jnp.sum(x, axis=1, keepdims=True)
    s2 = jnp.sum(x * x, axis=1, keepdims=True)
    mean = s1 * (1.0 / HW)
    var = s2 * (1.0 / HW) - mean * mean

    aff = lax.rsqrt(var + _EPS) * _INV_DENOM
    boff = mean * aff + (_GMID + _GRID_MIN * _INV_DENOM)
    v = jnp.clip(x * aff - boff, -14.0, 14.0)
    t_up = jnp.exp(v + v)
    t_dn = jnp.exp(-(v + v))
    b_mid = jnp.exp(-(v * v))

    pad_ref[pl.ds(_GMID * C, C), pl.ds(PADL, HW)] = b_mid.astype(bdt)
    b = b_mid
    for k in range(1, _G - _GMID):
        b = (b * t_up) * math.exp(-(2 * k - 1))
        pad_ref[pl.ds((_GMID + k) * C, C), pl.ds(PADL, HW)] = b.astype(bdt)
    b = b_mid
    for k in range(1, _GMID + 1):
        b = (b * t_dn) * math.exp(-(2 * k - 1))
        pad_ref[pl.ds((_GMID - k) * C, C), pl.ds(PADL, HW)] = b.astype(bdt)

    pad_ref[pl.ds(CG, C), pl.ds(PADL, HW)] = silu.astype(bdt)

    data = pad_ref[pl.ds(0, CTOT), pl.ds(PADL - 128, HW + 256)]
    p = jnp.dot(w_ref[...], data,
                preferred_element_type=jnp.float32)
    accs = [jnp.zeros((COUT, HW), jnp.float32) for _ in range(3)]
    for ky in range(3):
        for kx in range(3):
            r0 = (ky * 3 + kx) * COUT
            c0 = 128 + (ky - 1) * W + (kx - 1)
            accs[kx] = accs[kx] + p[r0:r0 + COUT, c0:c0 + HW]
    return accs[1] + accs[0] * mlf + accs[2] * mrf


def _encoder_block_kernel(x_ref, w1_ref, w2_ref, o_ref, *pads,
                          H, W, C1, C2, PADL):
    HW = H * W
    CMAX = pads[0].shape[0]

    hz = jnp.zeros((CMAX, W + 1), pads[0].dtype)
    for ref in pads:
        ref[pl.ds(0, CMAX), pl.ds(PADL - (W + 1), W + 1)] = hz
        ref[pl.ds(0, CMAX), pl.ds(PADL + HW, W + 1)] = hz

    col = lax.broadcasted_iota(jnp.int32, (1, HW), 1) % W
    mlf = (col >= 1).astype(jnp.float32)
    mrf = (col < (W - 1)).astype(jnp.float32)

    ys = [_fastkan_layer(x_ref[i], w1_ref, pad, mlf, mrf,
                         H=H, W=W, C=C1, COUT=C2, PADL=PADL)
          for i, pad in enumerate(pads)]
    zs = [_fastkan_layer(y, w2_ref, pad, mlf, mrf,
                         H=H, W=W, C=C2, COUT=C2, PADL=PADL)
          for y, pad in zip(ys, pads)]
    for i, z in enumerate(zs):
        o_ref[i] = z.astype(o_ref.dtype)


def _permute_to_g_major(w, c):
    cg = c * _G
    sp = w[:, :, :cg].reshape(9, -1, c, _G)
    sp = jnp.transpose(sp, (0, 1, 3, 2)).reshape(9, -1, cg)
    wp = jnp.concatenate([sp, w[:, :, cg:]], axis=-1).astype(jnp.bfloat16)
    return wp.reshape(-1, wp.shape[-1])


def kernel(x_nchw, w1, w2):
    n, c1, hh, ww = x_nchw.shape
    c2 = w1.shape[1]
    ct1, ct2 = w1.shape[2], w2.shape[2]
    hw = hh * ww
    padl = ((ww + 1 + 127) // 128) * 128
    lpad = padl + hw + 128
    cmax = max(ct1, ct2)
    ips = 4 if n % 4 == 0 else (2 if n % 2 == 0 else 1)

    w1b = _permute_to_g_major(w1, c1)
    w2b = _permute_to_g_major(w2, c2)
    x_flat = x_nchw.reshape(n, c1, hw)

    body = functools.partial(_encoder_block_kernel,
                             H=hh, W=ww, C1=c1, C2=c2, PADL=padl)
    out = pl.pallas_call(
        body,
        out_shape=jax.ShapeDtypeStruct((n, c2, hw), jnp.float32),
        grid_spec=pltpu.PrefetchScalarGridSpec(
            num_scalar_prefetch=0,
            grid=(n // ips,),
            in_specs=[
                pl.BlockSpec((ips, c1, hw), lambda i: (i, 0, 0)),
                pl.BlockSpec((9 * c2, ct1), lambda i: (0, 0)),
                pl.BlockSpec((9 * c2, ct2), lambda i: (0, 0)),
            ],
            out_specs=pl.BlockSpec((ips, c2, hw), lambda i: (i, 0, 0)),
            scratch_shapes=[pltpu.VMEM((cmax, lpad), jnp.bfloat16)
                            for _ in range(ips)],
        ),
        compiler_params=pltpu.CompilerParams(
            dimension_semantics=("parallel",)),
    )(x_flat, w1b, w2b)
    return out.reshape(n, c2, hh, ww)


# speedup vs baseline: 2.6002x; 1.0044x over previous
import functools
import math

import jax
import jax.numpy as jnp
from jax import lax
from jax.experimental import pallas as pl
from jax.experimental.pallas import tpu as pltpu

_G = 8
_GRID_MIN, _GRID_MAX = -2.0, 2.0
_INV_DENOM = (_G - 1) / (_GRID_MAX - _GRID_MIN)
_EPS = 1e-5
_GMID = _G // 2


def _fastkan_layer(x, w_ref, pad_ref, mlf, mrf, *, H, W, C, COUT, PADL):
    HW = H * W
    CG = C * _G
    CTOT = CG + C
    bdt = pad_ref.dtype

    xh = 0.5 * x
    silu = xh + xh * jnp.tanh(xh)

    s1 = jnp.sum(x, axis=1, keepdims=True)
    s2 = jnp.sum(x * x, axis=1, keepdims=True)
    mean = s1 * (1.0 / HW)
    var = s2 * (1.0 / HW) - mean * mean

    aff = lax.rsqrt(var + _EPS) * _INV_DENOM
    boff = mean * aff + (_GMID + _GRID_MIN * _INV_DENOM)
    v = jnp.clip(x * aff - boff, -14.0, 14.0)
    t_up = jnp.exp(v + v)
    t_dn = jnp.exp(-(v + v))
    b_mid = jnp.exp(-(v * v))

    pad_ref[pl.ds(_GMID * C, C), pl.ds(PADL, HW)] = b_mid.astype(bdt)
    b = b_mid
    for k in range(1, _G - _GMID):
        b = (b * t_up) * math.exp(-(2 * k - 1))
        pad_ref[pl.ds((_GMID + k) * C, C), pl.ds(PADL, HW)] = b.astype(bdt)
    b = b_mid
    for k in range(1, _GMID + 1):
        b = (b * t_dn) * math.exp(-(2 * k - 1))
        pad_ref[pl.ds((_GMID - k) * C, C), pl.ds(PADL, HW)] = b.astype(bdt)

    pad_ref[pl.ds(CG, C), pl.ds(PADL, HW)] = silu.astype(bdt)

    data = pad_ref[pl.ds(0, CTOT), pl.ds(PADL - 128, HW + 256)]
    p = jnp.dot(w_ref[...], data,
                preferred_element_type=jnp.float32)
    accs = [None, None, None]
    for ky in range(3):
        for kx in range(3):
            r0 = (ky * 3 + kx) * COUT
            c0 = 128 + (ky - 1) * W + (kx - 1)
            s = p[r0:r0 + COUT, c0:c0 + HW]
            accs[kx] = s if accs[kx] is None else accs[kx] + s
    return accs[1] + accs[0] * mlf + accs[2] * mrf


def _encoder_block_kernel(x_ref, w1_ref, w2_ref, o_ref, *pads,
                          H, W, C1, C2, PADL):
    HW = H * W
    CMAX = pads[0].shape[0]

    hz = jnp.zeros((CMAX, W + 1), pads[0].dtype)
    for ref in pads:
        ref[pl.ds(0, CMAX), pl.ds(PADL - (W + 1), W + 1)] = hz
        ref[pl.ds(0, CMAX), pl.ds(PADL + HW, W + 1)] = hz

    col = lax.broadcasted_iota(jnp.int32, (1, HW), 1) % W
    mlf = (col >= 1).astype(jnp.float32)
    mrf = (col < (W - 1)).astype(jnp.float32)

    ys = [_fastkan_layer(x_ref[i], w1_ref, pad, mlf, mrf,
                         H=H, W=W, C=C1, COUT=C2, PADL=PADL)
          for i, pad in enumerate(pads)]
    zs = [_fastkan_layer(y, w2_ref, pad, mlf, mrf,
                         H=H, W=W, C=C2, COUT=C2, PADL=PADL)
          for y, pad in zip(ys, pads)]
    for i, z in enumerate(zs):
        o_ref[i] = z.astype(o_ref.dtype)


def _permute_to_g_major(w, c):
    cg = c * _G
    sp = w[:, :, :cg].reshape(9, -1, c, _G)
    sp = jnp.transpose(sp, (0, 1, 3, 2)).reshape(9, -1, cg)
    wp = jnp.concatenate([sp, w[:, :, cg:]], axis=-1).astype(jnp.bfloat16)
    return wp.reshape(-1, wp.shape[-1])


def kernel(x_nchw, w1, w2):
    n, c1, hh, ww = x_nchw.shape
    c2 = w1.shape[1]
    ct1, ct2 = w1.shape[2], w2.shape[2]
    hw = hh * ww
    padl = ((ww + 1 + 127) // 128) * 128
    lpad = padl + hw + 128
    cmax = max(ct1, ct2)
    ips = 4 if n % 4 == 0 else (2 if n % 2 == 0 else 1)

    w1b = _permute_to_g_major(w1, c1)
    w2b = _permute_to_g_major(w2, c2)
    x_flat = x_nchw.reshape(n, c1, hw)

    body = functools.partial(_encoder_block_kernel,
                             H=hh, W=ww, C1=c1, C2=c2, PADL=padl)
    out = pl.pallas_call(
        body,
        out_shape=jax.ShapeDtypeStruct((n, c2, hw), jnp.float32),
        grid_spec=pltpu.PrefetchScalarGridSpec(
            num_scalar_prefetch=0,
            grid=(n // ips,),
            in_specs=[
                pl.BlockSpec((ips, c1, hw), lambda i: (i, 0, 0)),
                pl.BlockSpec((9 * c2, ct1), lambda i: (0, 0)),
                pl.BlockSpec((9 * c2, ct2), lambda i: (0, 0)),
            ],
            out_specs=pl.BlockSpec((ips, c2, hw), lambda i: (i, 0, 0)),
            scratch_shapes=[pltpu.VMEM((cmax, lpad), jnp.bfloat16)
                            for _ in range(ips)],
        ),
        compiler_params=pltpu.CompilerParams(
            dimension_semantics=("parallel",)),
    )(x_flat, w1b, w2b)
    return out.reshape(n, c2, hh, ww)
